# baseline (speedup 1.0000x reference)
import functools

import jax
import jax.numpy as jnp
from jax import lax
from jax.experimental import pallas as pl
from jax.experimental.pallas import tpu as pltpu
from jax.experimental.pallas import tpu_sc as plsc

_P = 8192
_INT_MIN = -2147483648
_CHUNK = 2048
_NSTEPS = 8
_PPS = 16384 // _NSTEPS
_RPS = 128 // _NSTEPS


def _key_body(coarse_ref, skey_ref):
    cb = coarse_ref[0]
    m1 = cb[0]
    m2 = jnp.full((128, 128), -jnp.inf, dtype=jnp.float32)
    for c in range(1, 19):
        v = cb[c]
        m2 = jnp.maximum(m2, jnp.minimum(v, m1))
        m1 = jnp.maximum(m1, v)
    unc = m2 - m1
    bits = lax.bitcast_convert_type(unc, jnp.int32)
    skey_ref[0] = jnp.where(bits >= 0, bits, jnp.int32(2147483647) - bits)


def _sc_mask_body(skey_hbm, mask_hbm, key_v, mask_v, hist_v, row_v, slab_v,
                  shared):
    cid = lax.axis_index("c")
    sid = lax.axis_index("s")
    img_local = sid // 8
    w = sid % 8
    img = cid * 2 + img_local

    lane = lax.iota(jnp.int32, 16)
    zeros16 = jnp.zeros((16,), jnp.float32)
    ones16 = jnp.ones((16,), jnp.float32)
    int_min_v = jnp.full((16,), _INT_MIN, jnp.int32)

    pltpu.sync_copy(skey_hbm.at[img, pl.ds(w * _CHUNK, _CHUNK)], key_v)

    prefix = jnp.int32(0)
    above = jnp.float32(0.0)
    pf = jnp.float32(_P)

    for r in range(8):
        shift = 28 - 4 * r
        for j in range(16):
            hist_v[pl.ds(j * 16, 16)] = zeros16

        def scan_body(i, carry, _r=r, _shift=shift, _prefix=prefix):
            kv = key_v[pl.ds(pl.multiple_of(i * 16, 16), 16)]
            uk = kv ^ int_min_v
            digit = lax.shift_right_logical(uk, jnp.int32(_shift)) & 15
            if _r == 0:
                act = digit >= 0
            else:
                act = lax.shift_right_logical(uk, jnp.int32(_shift + 4)) == _prefix
            plsc.addupdate_scatter(hist_v, [digit * 16 + lane], ones16,
                                   mask=act)
            return carry

        lax.fori_loop(0, _CHUNK // 16, scan_body, jnp.int32(0))

        tot = zeros16
        for l in range(16):
            tot = tot + plsc.load_gather(hist_v, [lane * 16 + l])
        row_v[...] = tot
        base = 32 + img_local * 16 + (r % 2) * 8
        pltpu.sync_copy(row_v, shared.at[base + w])
        plsc.subcore_barrier()
        pltpu.sync_copy(shared.at[pl.ds(base, 8)], slab_v)
        g = zeros16
        for j in range(8):
            g = g + slab_v[j]

        sfx_incl = lax.rev(plsc.cumsum(lax.rev(g, (0,))), (0,))
        strict = sfx_incl - g
        cond = ((above + strict) < pf) & ((above + strict + g) >= pf)
        condf = jnp.where(cond, 1.0, 0.0)
        dsel = jnp.sum(lane.astype(jnp.float32) * condf, axis=0)
        above = above + jnp.sum(strict * condf, axis=0)
        prefix = prefix * 16 + dsel.astype(jnp.int32)

    vstar = prefix ^ jnp.int32(_INT_MIN)
    vstar_v = jnp.zeros((16,), jnp.int32) + vstar
    r_slots = pf - above

    def tie_body(i, acc):
        kv = key_v[pl.ds(pl.multiple_of(i * 16, 16), 16)]
        return acc + jnp.sum(jnp.where(kv == vstar_v, 1.0, 0.0), axis=0)

    tcnt = lax.fori_loop(0, _CHUNK // 16, tie_body, jnp.float32(0.0))
    row_v[...] = zeros16 + tcnt
    tbase = 32 + img_local * 16
    pltpu.sync_copy(row_v, shared.at[tbase + w])
    plsc.subcore_barrier()
    pltpu.sync_copy(shared.at[pl.ds(tbase, 8)], slab_v)
    my_off_v = zeros16
    for j in range(8):
        my_off_v = my_off_v + slab_v[j] * jnp.where(jnp.int32(j) < w, 1.0, 0.0)
    my_off = jnp.max(my_off_v, axis=0)

    def mask_body(i, off):
        kv = key_v[pl.ds(pl.multiple_of(i * 16, 16), 16)]
        tie = kv == vstar_v
        tf = jnp.where(tie, 1.0, 0.0)
        excl = plsc.cumsum(tf) - tf
        sel = tie & ((excl + off) < r_slots)
        m = (kv > vstar_v) | sel
        mask_v[pl.ds(pl.multiple_of(i * 16, 16), 16)] = jnp.where(m, 1.0, 0.0)
        return off + jnp.sum(tf, axis=0)

    lax.fori_loop(0, _CHUNK // 16, mask_body, my_off)
    pltpu.sync_copy(mask_v, mask_hbm.at[img, pl.ds(w * _CHUNK, _CHUNK)])


def _select_body(logits_ref, coarse_ref, mask_ref, out_ref):
    mrow = mask_ref[0, 0, 0]
    out_ref[0] = jnp.where(mrow[None, :] > 0.5, logits_ref[0], coarse_ref[0])


def _mlp_body(fine_ref, coarse_ref, w1f_ref, w1c_ref, b1_ref,
              w2_ref, b2_ref, w3_ref, b3_ref, out_ref, pooled_s):
    f = fine_ref[0]
    w_i = lax.broadcasted_iota(jnp.int32, (256, 128), 0)
    x_i = lax.broadcasted_iota(jnp.int32, (256, 128), 1)
    pool = ((lax.shift_right_logical(w_i, 1) == x_i)
            .astype(jnp.float32) * 0.25).astype(jnp.bfloat16)
    for j in range(_RPS):
        s = f[:, 2 * j, :] + f[:, 2 * j + 1, :]
        p = jnp.dot(s.astype(jnp.bfloat16), pool,
                    preferred_element_type=jnp.float32)
        pooled_s[:, j * 128:(j + 1) * 128] = p.astype(jnp.bfloat16)
    pooled = pooled_s[...]
    call = coarse_ref[0]
    h1 = jnp.dot(w1f_ref[...], pooled, preferred_element_type=jnp.float32)
    h1 = h1 + jnp.dot(w1c_ref[...], call.astype(jnp.bfloat16),
                      preferred_element_type=jnp.float32) + b1_ref[...]
    h1 = jnp.maximum(h1, 0.0).astype(jnp.bfloat16)
    h2 = jnp.dot(w2_ref[...], h1, preferred_element_type=jnp.float32)
    h2 = jnp.maximum(h2 + b2_ref[...], 0.0).astype(jnp.bfloat16)
    o = jnp.dot(w3_ref[...], h2, preferred_element_type=jnp.float32)
    out_ref[0] = o + b3_ref[...]


@jax.jit
def kernel(coarse_logits, fine_features, w1, b1, w2, b2, w3, b3):
    N, C, H, W = coarse_logits.shape
    skey = pl.pallas_call(
        _key_body,
        grid=(N,),
        in_specs=[pl.BlockSpec((1, C, H, W), lambda n: (n, 0, 0, 0))],
        out_specs=pl.BlockSpec((1, H, W), lambda n: (n, 0, 0)),
        out_shape=jax.ShapeDtypeStruct((N, H, W), jnp.int32),
    )(coarse_logits)

    mesh = plsc.VectorSubcoreMesh(core_axis_name="c", subcore_axis_name="s")
    mask_flat = pl.kernel(
        _sc_mask_body,
        out_type=jax.ShapeDtypeStruct((N, H * W), jnp.float32),
        mesh=mesh,
        compiler_params=pltpu.CompilerParams(needs_layout_passes=False),
        scratch_types=[
            pltpu.VMEM((_CHUNK,), jnp.int32),
            pltpu.VMEM((_CHUNK,), jnp.float32),
            pltpu.VMEM((256,), jnp.float32),
            pltpu.VMEM((16,), jnp.float32),
            pltpu.VMEM((8, 16), jnp.float32),
            pltpu.VMEM_SHARED((64, 16), jnp.float32),
        ],
    )(skey.reshape(N, H * W))

    w1f = w1[:, :256].astype(jnp.bfloat16)
    w1c = w1[:, 256:].astype(jnp.bfloat16)
    b1c = b1[:, None]
    b2c = b2[:, None]
    b3c = b3[:, None]
    w2b = w2.astype(jnp.bfloat16)
    w3b = w3.astype(jnp.bfloat16)

    coarse2 = coarse_logits.reshape(N, C, H * W)
    const = lambda n, i: (0, 0)
    logits = pl.pallas_call(
        _mlp_body,
        grid=(N, _NSTEPS),
        in_specs=[
            pl.BlockSpec((1, 256, 2 * _RPS, 256), lambda n, i: (n, 0, i, 0)),
            pl.BlockSpec((1, C, _PPS), lambda n, i: (n, 0, i)),
            pl.BlockSpec((512, 256), const),
            pl.BlockSpec((512, 19), const),
            pl.BlockSpec((512, 1), const),
            pl.BlockSpec((512, 512), const),
            pl.BlockSpec((512, 1), const),
            pl.BlockSpec((19, 512), const),
            pl.BlockSpec((19, 1), const),
        ],
        out_specs=pl.BlockSpec((1, C, _PPS), lambda n, i: (n, 0, i)),
        out_shape=jax.ShapeDtypeStruct((N, C, H * W), jnp.float32),
        scratch_shapes=[pltpu.VMEM((256, _PPS), jnp.bfloat16)],
    )(fine_features, coarse2, w1f, w1c, b1c, w2b, b2c, w3b, b3c)

    nsel = 16
    mask4 = mask_flat.reshape(N, nsel, 1, (H * W) // nsel)
    psel = (H * W) // nsel
    refined = pl.pallas_call(
        _select_body,
        grid=(N, nsel),
        in_specs=[
            pl.BlockSpec((1, C, psel), lambda n, i: (n, 0, i)),
            pl.BlockSpec((1, C, psel), lambda n, i: (n, 0, i)),
            pl.BlockSpec((1, 1, 1, psel), lambda n, i: (n, i, 0, 0)),
        ],
        out_specs=pl.BlockSpec((1, C, psel), lambda n, i: (n, 0, i)),
        out_shape=jax.ShapeDtypeStruct((N, C, H * W), jnp.float32),
    )(logits, coarse2, mask4)
    return refined.reshape(N, C, H, W)


# speedup vs baseline: 1.1341x; 1.1341x over previous
import functools

import jax
import jax.numpy as jnp
from jax import lax
from jax.experimental import pallas as pl
from jax.experimental.pallas import tpu as pltpu
from jax.experimental.pallas import tpu_sc as plsc

_P = 8192
_INT_MIN = -2147483648
_CHUNK = 2048
_NSTEPS = 8
_PPS = 16384 // _NSTEPS
_RPS = 128 // _NSTEPS


def _key_body(coarse_ref, skey_ref):
    cb = coarse_ref[0]
    m1 = cb[0]
    m2 = jnp.full((128, 128), -jnp.inf, dtype=jnp.float32)
    for c in range(1, 19):
        v = cb[c]
        m2 = jnp.maximum(m2, jnp.minimum(v, m1))
        m1 = jnp.maximum(m1, v)
    unc = m2 - m1
    bits = lax.bitcast_convert_type(unc, jnp.int32)
    skey_ref[0] = jnp.where(bits >= 0, bits, jnp.int32(2147483647) - bits)


def _sc_mask_body(skey_hbm, mask_hbm, key_v, mask_v, hist_v, row_v, slab_v,
                  shared):
    cid = lax.axis_index("c")
    sid = lax.axis_index("s")
    img_local = sid // 8
    w = sid % 8
    img = cid * 2 + img_local

    lane = lax.iota(jnp.int32, 16)
    zeros16 = jnp.zeros((16,), jnp.float32)
    ones16 = jnp.ones((16,), jnp.float32)
    int_min_v = jnp.full((16,), _INT_MIN, jnp.int32)

    pltpu.sync_copy(skey_hbm.at[img, pl.ds(w * _CHUNK, _CHUNK)], key_v)

    prefix = jnp.int32(0)
    above = jnp.float32(0.0)
    pf = jnp.float32(_P)

    for r in range(8):
        shift = 28 - 4 * r
        for j in range(16):
            hist_v[pl.ds(j * 16, 16)] = zeros16

        def scan_body(i, carry, _r=r, _shift=shift, _prefix=prefix):
            kv = key_v[pl.ds(pl.multiple_of(i * 16, 16), 16)]
            uk = kv ^ int_min_v
            digit = lax.shift_right_logical(uk, jnp.int32(_shift)) & 15
            if _r == 0:
                act = digit >= 0
            else:
                act = lax.shift_right_logical(uk, jnp.int32(_shift + 4)) == _prefix
            plsc.addupdate_scatter(hist_v, [digit * 16 + lane], ones16,
                                   mask=act)
            return carry

        lax.fori_loop(0, _CHUNK // 16, scan_body, jnp.int32(0))

        tot = zeros16
        for l in range(16):
            tot = tot + plsc.load_gather(hist_v, [lane * 16 + l])
        row_v[...] = tot
        base = 32 + img_local * 16 + (r % 2) * 8
        pltpu.sync_copy(row_v, shared.at[base + w])
        plsc.subcore_barrier()
        pltpu.sync_copy(shared.at[pl.ds(base, 8)], slab_v)
        g = zeros16
        for j in range(8):
            g = g + slab_v[j]

        sfx_incl = lax.rev(plsc.cumsum(lax.rev(g, (0,))), (0,))
        strict = sfx_incl - g
        cond = ((above + strict) < pf) & ((above + strict + g) >= pf)
        condf = jnp.where(cond, 1.0, 0.0)
        dsel = jnp.sum(lane.astype(jnp.float32) * condf, axis=0)
        above = above + jnp.sum(strict * condf, axis=0)
        prefix = prefix * 16 + dsel.astype(jnp.int32)

    vstar = prefix ^ jnp.int32(_INT_MIN)
    vstar_v = jnp.zeros((16,), jnp.int32) + vstar
    r_slots = pf - above

    def tie_body(i, acc):
        kv = key_v[pl.ds(pl.multiple_of(i * 16, 16), 16)]
        return acc + jnp.sum(jnp.where(kv == vstar_v, 1.0, 0.0), axis=0)

    tcnt = lax.fori_loop(0, _CHUNK // 16, tie_body, jnp.float32(0.0))
    row_v[...] = zeros16 + tcnt
    tbase = 32 + img_local * 16
    pltpu.sync_copy(row_v, shared.at[tbase + w])
    plsc.subcore_barrier()
    pltpu.sync_copy(shared.at[pl.ds(tbase, 8)], slab_v)
    my_off_v = zeros16
    for j in range(8):
        my_off_v = my_off_v + slab_v[j] * jnp.where(jnp.int32(j) < w, 1.0, 0.0)
    my_off = jnp.max(my_off_v, axis=0)

    def mask_body(i, off):
        kv = key_v[pl.ds(pl.multiple_of(i * 16, 16), 16)]
        tie = kv == vstar_v
        tf = jnp.where(tie, 1.0, 0.0)
        excl = plsc.cumsum(tf) - tf
        sel = tie & ((excl + off) < r_slots)
        m = (kv > vstar_v) | sel
        mask_v[pl.ds(pl.multiple_of(i * 16, 16), 16)] = jnp.where(m, 1.0, 0.0)
        return off + jnp.sum(tf, axis=0)

    lax.fori_loop(0, _CHUNK // 16, mask_body, my_off)
    pltpu.sync_copy(mask_v, mask_hbm.at[img, pl.ds(w * _CHUNK, _CHUNK)])


def _mlp_body(fine_ref, coarse_ref, mask_ref, w1f_ref, w1c_ref, b1_ref,
              w2_ref, b2_ref, w3_ref, b3_ref, out_ref, pooled_s):
    f = fine_ref[0]
    w_i = lax.broadcasted_iota(jnp.int32, (256, 128), 0)
    x_i = lax.broadcasted_iota(jnp.int32, (256, 128), 1)
    pool = ((lax.shift_right_logical(w_i, 1) == x_i)
            .astype(jnp.float32) * 0.25).astype(jnp.bfloat16)
    for j in range(_RPS):
        s = f[:, 2 * j, :] + f[:, 2 * j + 1, :]
        p = jnp.dot(s.astype(jnp.bfloat16), pool,
                    preferred_element_type=jnp.float32)
        pooled_s[:, j * 128:(j + 1) * 128] = p.astype(jnp.bfloat16)
    pooled = pooled_s[...]
    call = coarse_ref[0]
    h1 = jnp.dot(w1f_ref[...], pooled, preferred_element_type=jnp.float32)
    h1 = h1 + jnp.dot(w1c_ref[...], call.astype(jnp.bfloat16),
                      preferred_element_type=jnp.float32) + b1_ref[...]
    h1 = jnp.maximum(h1, 0.0).astype(jnp.bfloat16)
    h2 = jnp.dot(w2_ref[...], h1, preferred_element_type=jnp.float32)
    h2 = jnp.maximum(h2 + b2_ref[...], 0.0).astype(jnp.bfloat16)
    o = jnp.dot(w3_ref[...], h2, preferred_element_type=jnp.float32)
    o = o + b3_ref[...]
    mrow = mask_ref[0, 0, 0]
    out_ref[0] = jnp.where(mrow[None, :] > 0.5, o, call)


@jax.jit
def kernel(coarse_logits, fine_features, w1, b1, w2, b2, w3, b3):
    N, C, H, W = coarse_logits.shape
    skey = pl.pallas_call(
        _key_body,
        grid=(N,),
        in_specs=[pl.BlockSpec((1, C, H, W), lambda n: (n, 0, 0, 0))],
        out_specs=pl.BlockSpec((1, H, W), lambda n: (n, 0, 0)),
        out_shape=jax.ShapeDtypeStruct((N, H, W), jnp.int32),
    )(coarse_logits)

    mesh = plsc.VectorSubcoreMesh(core_axis_name="c", subcore_axis_name="s")
    mask_flat = pl.kernel(
        _sc_mask_body,
        out_type=jax.ShapeDtypeStruct((N, H * W), jnp.float32),
        mesh=mesh,
        compiler_params=pltpu.CompilerParams(needs_layout_passes=False),
        scratch_types=[
            pltpu.VMEM((_CHUNK,), jnp.int32),
            pltpu.VMEM((_CHUNK,), jnp.float32),
            pltpu.VMEM((256,), jnp.float32),
            pltpu.VMEM((16,), jnp.float32),
            pltpu.VMEM((8, 16), jnp.float32),
            pltpu.VMEM_SHARED((64, 16), jnp.float32),
        ],
    )(skey.reshape(N, H * W))

    w1f = w1[:, :256].astype(jnp.bfloat16)
    w1c = w1[:, 256:].astype(jnp.bfloat16)
    b1c = b1[:, None]
    b2c = b2[:, None]
    b3c = b3[:, None]
    w2b = w2.astype(jnp.bfloat16)
    w3b = w3.astype(jnp.bfloat16)

    coarse2 = coarse_logits.reshape(N, C, H * W)
    mask4 = mask_flat.reshape(N, _NSTEPS, 1, _PPS)
    const = lambda n, i: (0, 0)
    refined = pl.pallas_call(
        _mlp_body,
        grid=(N, _NSTEPS),
        in_specs=[
            pl.BlockSpec((1, 256, 2 * _RPS, 256), lambda n, i: (n, 0, i, 0)),
            pl.BlockSpec((1, C, _PPS), lambda n, i: (n, 0, i)),
            pl.BlockSpec((1, 1, 1, _PPS), lambda n, i: (n, i, 0, 0)),
            pl.BlockSpec((512, 256), const),
            pl.BlockSpec((512, 19), const),
            pl.BlockSpec((512, 1), const),
            pl.BlockSpec((512, 512), const),
            pl.BlockSpec((512, 1), const),
            pl.BlockSpec((19, 512), const),
            pl.BlockSpec((19, 1), const),
        ],
        out_specs=pl.BlockSpec((1, C, _PPS), lambda n, i: (n, 0, i)),
        out_shape=jax.ShapeDtypeStruct((N, C, H * W), jnp.float32),
        scratch_shapes=[pltpu.VMEM((256, _PPS), jnp.bfloat16)],
    )(fine_features, coarse2, mask4, w1f, w1c, b1c, w2b, b2c, w3b, b3c)
    return refined.reshape(N, C, H, W)
